# baseline (speedup 1.0000x reference)
import functools

import jax
import jax.numpy as jnp
from jax import lax
from jax.experimental import pallas as pl
from jax.experimental.pallas import tpu as pltpu
from jax.experimental.pallas import tpu_sc as plsc

NNODES = 10000
NREL = 16
HID = 64
NEDGE = 160000
NTGT = 1024
NCORE = 2
NSUB = 16
NWORK = NCORE * NSUB
EPAD = 163840
EPW = EPAD // NWORK
EPW1 = EPAD // NSUB
MCHUNK = 1024
CCHUNK = 1024
NSTRIPE = 624
NSTRIPE_LAST = NNODES - (NSUB - 1) * NSTRIPE
TPW = 2 * NTGT // NWORK

_mesh = plsc.VectorSubcoreMesh(
    core_axis_name="c", subcore_axis_name="s", num_cores=NCORE, num_subcores=NSUB
)
_sc_params = pltpu.CompilerParams(use_tc_tiling_on_sc=False, needs_layout_passes=False)


def _wid():
    return lax.axis_index("s") * NCORE + lax.axis_index("c")


def _striped(fn):
    s = lax.axis_index("s")

    @pl.when(s < NSUB - 1)
    def _():
        fn(pl.multiple_of(s * NSTRIPE, 8), NSTRIPE)

    @pl.when(s == NSUB - 1)
    def _():
        fn((NSUB - 1) * NSTRIPE, NSTRIPE_LAST)


@functools.partial(
    pl.kernel,
    out_type=jax.ShapeDtypeStruct((NNODES, NREL), jnp.float32),
    mesh=_mesh,
    compiler_params=_sc_params,
    scratch_types=[
        pltpu.VMEM((CCHUNK, NREL), jnp.float32),
        pltpu.VMEM((EPW1 // 128, 128), jnp.int32),
        pltpu.VMEM((EPW1,), jnp.int32),
        pltpu.VMEM_SHARED((NNODES, NREL), jnp.float32),
    ],
)
def _sc_counts(dst_hbm, et_hbm, zeros_hbm, out_hbm, oh_v, dst_v, et_v, shared):
    c = lax.axis_index("c")
    s = lax.axis_index("s")

    @pl.when(c == 0)
    def _():
        _striped(lambda off, n: pltpu.sync_copy(
            zeros_hbm.at[pl.ds(off, n)], shared.at[pl.ds(off, n)]))
        base = pl.multiple_of(s * EPW1, EPW1)
        pltpu.sync_copy(
            dst_hbm.at[pl.ds(pl.multiple_of(base // 128, 8), EPW1 // 128)], dst_v)
        pltpu.sync_copy(et_hbm.at[pl.ds(base, EPW1)], et_v)
        iota16 = lax.iota(jnp.int32, 16)
        ones16 = jnp.ones((16,), jnp.float32)
        zeros16 = jnp.zeros((16,), jnp.float32)

        def zrow(i, cc):
            oh_v[i, :] = zeros16
            return cc

        lax.fori_loop(0, CCHUNK, zrow, 0)
        plsc.subcore_barrier()

        def chunk(k, carry):
            def mark(g, cc):
                i_vec = g * 16 + iota16
                et16 = et_v[pl.ds(k * CCHUNK + g * 16, 16)]
                mask = base + k * CCHUNK + i_vec < NEDGE
                plsc.store_scatter(oh_v, [i_vec, et16], ones16, mask=mask)
                return cc

            lax.fori_loop(0, CCHUNK // 16, mark, 0)
            for j in range(CCHUNK // 128):
                pltpu.sync_copy(
                    oh_v.at[pl.ds(j * 128, 128)],
                    shared.at[dst_v.at[pl.multiple_of(k * (CCHUNK // 128), 8) + j]],
                    add=True)

            def unmark(g, cc):
                i_vec = g * 16 + iota16
                et16 = et_v[pl.ds(k * CCHUNK + g * 16, 16)]
                plsc.store_scatter(oh_v, [i_vec, et16], zeros16)
                return cc

            lax.fori_loop(0, CCHUNK // 16, unmark, 0)
            return carry

        lax.fori_loop(0, EPW1 // CCHUNK, chunk, 0)
        plsc.subcore_barrier()
        _striped(lambda off, n: pltpu.sync_copy(
            shared.at[pl.ds(off, n)], out_hbm.at[pl.ds(off, n)]))


MSEG = 128
HALF = 256
SEGH = HALF // MSEG
NH = EPW // HALF
NWAVE = 8
WED = EPW // NWAVE

_MSG_SCRATCH = [
    pltpu.VMEM((EPW // 128, 128), jnp.int32),
    pltpu.VMEM((EPW // 128, 128), jnp.int32),
    pltpu.VMEM((EPW,), jnp.float32),
    pltpu.VMEM((HALF, HID), jnp.float32),
    pltpu.VMEM((HALF, HID), jnp.float32),
    pltpu.VMEM_SHARED((NNODES, HID), jnp.float32),
    pltpu.SemaphoreType.DMA,
    pltpu.SemaphoreType.DMA,
]


def _msg_pipeline(ytab_hbm, gidx_v, dst_v, norm_v, rows_a, rows_b,
                  shared, sem_a, sem_b):
    bufs = (rows_a, rows_b)
    sems = (sem_a, sem_b)

    def fire(t):
        buf = bufs[t % 2]
        sem = sems[t % 2]
        return [
            pltpu.async_copy(
                ytab_hbm.at[gidx_v.at[t * SEGH + j]],
                buf.at[pl.ds(j * MSEG, MSEG)], sem)
            for j in range(SEGH)
        ]

    pending = fire(0)
    for t in range(NH):
        for d in pending:
            d.wait()
        if t + 1 < NH:
            pending = fire(t + 1)
        buf = bufs[t % 2]

        def grp(g, cc):
            nv16 = norm_v[pl.ds(t * HALF + g * 16, 16)]
            i0 = g * 16
            for l in range(16):
                nv = jnp.broadcast_to(nv16[l], (16,))
                for q in range(HID // 16):
                    sl = pl.ds(q * 16, 16)
                    buf[i0 + l, sl] = buf[i0 + l, sl] * nv
            return cc

        lax.fori_loop(0, HALF // 16, grp, 0)
        for j in range(SEGH):
            pltpu.sync_copy(
                buf.at[pl.ds(j * MSEG, MSEG)],
                shared.at[dst_v.at[t * SEGH + j]], add=True)


@functools.partial(
    pl.kernel,
    out_type=[
        jax.ShapeDtypeStruct((NCORE, NNODES, HID), jnp.float32),
        jax.ShapeDtypeStruct((EPAD,), jnp.float32),
    ],
    mesh=_mesh,
    compiler_params=_sc_params,
    scratch_types=_MSG_SCRATCH + [
        pltpu.VMEM((WED, NREL), jnp.float32),
        pltpu.VMEM((WED,), jnp.int32),
    ],
)
def _sc_messages0(ytab_hbm, gidx_hbm, dst_hbm, et_hbm, c2d_hbm, zeros_hbm,
                  out_hbm, norm_out_hbm,
                  gidx_v, dst_v, norm_v, rows_a, rows_b, shared, sem_a, sem_b,
                  crows_v, et_v):
    c = lax.axis_index("c")
    w = _wid()
    _striped(lambda off, n: pltpu.sync_copy(
        zeros_hbm.at[pl.ds(off, n)], shared.at[pl.ds(off, n)]))
    base = pl.multiple_of(w * EPW, EPW)
    rbase = pl.multiple_of(base // 128, 8)
    pltpu.sync_copy(gidx_hbm.at[pl.ds(rbase, EPW // 128)], gidx_v)
    pltpu.sync_copy(dst_hbm.at[pl.ds(rbase, EPW // 128)], dst_v)
    iota16 = lax.iota(jnp.int32, 16)
    one = jnp.ones((16,), jnp.float32)
    for v in range(NWAVE):
        pltpu.sync_copy(et_hbm.at[pl.ds(base + v * WED, WED)], et_v)
        descs = [
            pltpu.async_copy(
                c2d_hbm.at[dst_v.at[v * (WED // 128) + j]],
                crows_v.at[pl.ds(j * 128, 128)], sem_a)
            for j in range(WED // 128)
        ]
        for d in descs:
            d.wait()

        def ngrp(g, cc):
            i_vec = g * 16 + iota16
            et16 = et_v[pl.ds(g * 16, 16)]
            cv = plsc.load_gather(crows_v, [i_vec, et16])
            inv = one / jnp.maximum(cv, one)
            inv = jnp.where(base + v * WED + i_vec < NEDGE, inv,
                            jnp.zeros((16,), jnp.float32))
            norm_v[pl.ds(v * WED + g * 16, 16)] = inv
            return cc

        lax.fori_loop(0, WED // 16, ngrp, 0)
    pltpu.sync_copy(norm_v, norm_out_hbm.at[pl.ds(base, EPW)])
    plsc.subcore_barrier()
    _msg_pipeline(ytab_hbm, gidx_v, dst_v, norm_v, rows_a, rows_b,
                  shared, sem_a, sem_b)
    plsc.subcore_barrier()
    _striped(lambda off, n: pltpu.sync_copy(
        shared.at[pl.ds(off, n)], out_hbm.at[c].at[pl.ds(off, n)]))


@functools.partial(
    pl.kernel,
    out_type=[
        jax.ShapeDtypeStruct((NCORE, NNODES, HID), jnp.float32),
        jax.ShapeDtypeStruct((3, 2 * NTGT, HID), jnp.float32),
    ],
    mesh=_mesh,
    compiler_params=_sc_params,
    scratch_types=_MSG_SCRATCH + [
        pltpu.VMEM((128,), jnp.int32),
        pltpu.VMEM((TPW,), jnp.int32),
    ],
)
def _sc_messages1(ytab_hbm, gidx_hbm, dst_hbm, norm_hbm, hr_hbm, tgt_hbm,
                  zeros_hbm, out_hbm, g3_hbm,
                  gidx_v, dst_v, norm_v, rows_a, rows_b, shared, sem_a, sem_b,
                  idxc_v, idxh_v):
    c = lax.axis_index("c")
    s = lax.axis_index("s")
    w = _wid()
    _striped(lambda off, n: pltpu.sync_copy(
        zeros_hbm.at[pl.ds(off, n)], shared.at[pl.ds(off, n)]))
    base = pl.multiple_of(w * EPW, EPW)
    rbase = pl.multiple_of(base // 128, 8)
    pltpu.sync_copy(gidx_hbm.at[pl.ds(rbase, EPW // 128)], gidx_v)
    pltpu.sync_copy(dst_hbm.at[pl.ds(rbase, EPW // 128)], dst_v)
    pltpu.sync_copy(norm_hbm.at[pl.ds(base, EPW)], norm_v)
    plsc.subcore_barrier()
    _msg_pipeline(ytab_hbm, gidx_v, dst_v, norm_v, rows_a, rows_b,
                  shared, sem_a, sem_b)
    plsc.subcore_barrier()
    _striped(lambda off, n: pltpu.sync_copy(
        shared.at[pl.ds(off, n)], out_hbm.at[c].at[pl.ds(off, n)]))
    plsc.subcore_barrier()
    tb = pl.multiple_of(s * 128, 8)
    pltpu.sync_copy(tgt_hbm.at[pl.ds(tb, 128)], idxc_v)
    pltpu.async_copy(
        out_hbm.at[c].at[idxc_v], rows_a.at[pl.ds(0, 128)], sem_a).wait()
    pltpu.sync_copy(rows_a.at[pl.ds(0, 128)], g3_hbm.at[c].at[pl.ds(tb, 128)])
    hb = pl.multiple_of(w * TPW, 8)
    pltpu.sync_copy(tgt_hbm.at[pl.ds(hb, TPW)], idxh_v)
    pltpu.async_copy(hr_hbm.at[idxh_v], rows_b.at[pl.ds(0, TPW)], sem_b).wait()
    pltpu.sync_copy(rows_b.at[pl.ds(0, TPW)], g3_hbm.at[2].at[pl.ds(hb, TPW)])


def _mm_body(a_ref, w_ref, y_ref, r_ref):
    p = jnp.dot(a_ref[...], w_ref[...], preferred_element_type=jnp.float32)
    r_ref[...] = p[:, :HID]
    for r in range(NREL):
        y_ref[r] = p[:, 2 * HID + r * HID:2 * HID + (r + 1) * HID]


def _tc_matmul(h, wcat):
    m, kdim = h.shape
    ncols = wcat.shape[1]
    bm = 400
    return pl.pallas_call(
        _mm_body,
        grid=(m // bm,),
        in_specs=[
            pl.BlockSpec((bm, kdim), lambda i: (i, 0)),
            pl.BlockSpec((kdim, ncols), lambda i: (0, 0)),
        ],
        out_specs=[
            pl.BlockSpec((NREL, bm, HID), lambda i: (0, i, 0)),
            pl.BlockSpec((bm, HID), lambda i: (i, 0)),
        ],
        out_shape=[
            jax.ShapeDtypeStruct((NREL, m, HID), jnp.float32),
            jax.ShapeDtypeStruct((m, HID), jnp.float32),
        ],
    )(h, wcat)


def _mm_fused_body(p0_ref, p1_ref, hr_ref, b_ref, w_ref, y_ref, r_ref):
    a = jnp.maximum(p0_ref[...] + p1_ref[...] + hr_ref[...] + b_ref[...], 0.0)
    p = jnp.dot(a, w_ref[...], preferred_element_type=jnp.float32)
    r_ref[...] = p[:, :HID]
    for r in range(NREL):
        y_ref[r] = p[:, 2 * HID + r * HID:2 * HID + (r + 1) * HID]


def _tc_matmul_fused(p0, p1, hr, bias, wcat):
    m = p0.shape[0]
    ncols = wcat.shape[1]
    bm = 400
    spec = pl.BlockSpec((bm, HID), lambda i: (i, 0))
    return pl.pallas_call(
        _mm_fused_body,
        grid=(m // bm,),
        in_specs=[
            spec, spec, spec,
            pl.BlockSpec((1, HID), lambda i: (0, 0)),
            pl.BlockSpec((HID, ncols), lambda i: (0, 0)),
        ],
        out_specs=[
            pl.BlockSpec((NREL, bm, HID), lambda i: (0, i, 0)),
            pl.BlockSpec((bm, HID), lambda i: (i, 0)),
        ],
        out_shape=[
            jax.ShapeDtypeStruct((NREL, m, HID), jnp.float32),
            jax.ShapeDtypeStruct((m, HID), jnp.float32),
        ],
    )(p0, p1, hr, bias.reshape(1, HID), wcat)


def _dec_body(g_ref, b_ref, w2_ref, s_ref, o_ref):
    h = jnp.maximum(g_ref[0] + g_ref[1] + g_ref[2] + b_ref[...], 0.0)
    x1 = h[:NTGT]
    x2 = h[NTGT:]
    p = jnp.dot(x1, w2_ref[...], preferred_element_type=jnp.float32)
    x2t = jnp.concatenate([x2] * NREL, axis=1)
    rec = jnp.dot(p * x2t, s_ref[...], preferred_element_type=jnp.float32)
    o_ref[...] = jax.nn.sigmoid(rec)


def _tc_decoder(g3, bias, w2cat, sel):
    return pl.pallas_call(
        _dec_body,
        out_shape=jax.ShapeDtypeStruct((NTGT, NREL), jnp.float32),
    )(g3, bias.reshape(1, HID), w2cat, sel)


def kernel(x, edge_index, edge_attr, target_edge_index,
           bases0, comp0, root0, bias0, bases1, comp1, root1, bias1, R, D):
    f32 = jnp.float32
    et = jnp.argmax(edge_attr, axis=1).astype(jnp.int32)
    src = edge_index[0]
    dst = edge_index[1]
    gidx = et * NNODES + src
    pad = EPAD - NEDGE
    gidx_p = jnp.pad(gidx, (0, pad)).reshape(EPAD // 128, 128)
    dst_p = jnp.pad(dst, (0, pad)).reshape(EPAD // 128, 128)
    et_p = jnp.pad(et, (0, pad))
    zeros_n16 = jnp.zeros((NNODES, NREL), f32)
    zeros_nh = jnp.zeros((NNODES, HID), f32)

    c2d = _sc_counts(dst_p, et_p, zeros_n16)

    def wcat(comp, bases, root):
        indim = root.shape[0]
        w = jnp.einsum('rb,bio->rio', comp, bases)
        w = w.transpose(1, 0, 2).reshape(indim, NREL * HID)
        return jnp.concatenate([root, jnp.zeros((indim, HID), f32), w], axis=1)

    y0, xr0 = _tc_matmul(x, wcat(comp0, bases0, root0))
    ap0, norm = _sc_messages0(y0.reshape(NREL * NNODES, HID), gidx_p, dst_p,
                              et_p, c2d, zeros_nh)

    y1, hr1 = _tc_matmul_fused(ap0[0], ap0[1], xr0, bias0,
                               wcat(comp1, bases1, root1))
    tgt = target_edge_index.reshape(-1).astype(jnp.int32)
    _, g3 = _sc_messages1(y1.reshape(NREL * NNODES, HID), gidx_p, dst_p, norm,
                          hr1, tgt, zeros_nh)
    w2 = (D[:, :, None] * R[None, :, :] * D[:, None, :])
    w2 = w2.transpose(1, 0, 2).reshape(HID, NREL * HID)
    sel = (lax.broadcasted_iota(jnp.int32, (NREL * HID, NREL), 0) // HID ==
           lax.broadcasted_iota(jnp.int32, (NREL * HID, NREL), 1)).astype(f32)
    return _tc_decoder(g3, bias1, w2, sel)


# speedup vs baseline: 1.0777x; 1.0777x over previous
import functools

import jax
import jax.numpy as jnp
from jax import lax
from jax.experimental import pallas as pl
from jax.experimental.pallas import tpu as pltpu
from jax.experimental.pallas import tpu_sc as plsc

NNODES = 10000
NREL = 16
HID = 64
NEDGE = 160000
NTGT = 1024
NCORE = 2
NSUB = 16
NWORK = NCORE * NSUB
EPAD = 163840
EPW = EPAD // NWORK
EPW1 = EPAD // NSUB
MCHUNK = 1024
CCHUNK = 1024
NSTRIPE = 624
NSTRIPE_LAST = NNODES - (NSUB - 1) * NSTRIPE
TPW = 2 * NTGT // NWORK

_mesh = plsc.VectorSubcoreMesh(
    core_axis_name="c", subcore_axis_name="s", num_cores=NCORE, num_subcores=NSUB
)
_sc_params = pltpu.CompilerParams(use_tc_tiling_on_sc=False, needs_layout_passes=False)


def _wid():
    return lax.axis_index("s") * NCORE + lax.axis_index("c")


def _striped(fn):
    s = lax.axis_index("s")

    @pl.when(s < NSUB - 1)
    def _():
        fn(pl.multiple_of(s * NSTRIPE, 8), NSTRIPE)

    @pl.when(s == NSUB - 1)
    def _():
        fn((NSUB - 1) * NSTRIPE, NSTRIPE_LAST)


@functools.partial(
    pl.kernel,
    out_type=jax.ShapeDtypeStruct((NNODES, NREL), jnp.float32),
    mesh=_mesh,
    compiler_params=_sc_params,
    scratch_types=[
        pltpu.VMEM((CCHUNK, NREL), jnp.float32),
        pltpu.VMEM((EPW1 // 128, 128), jnp.int32),
        pltpu.VMEM((EPW1,), jnp.int32),
        pltpu.VMEM_SHARED((NNODES, NREL), jnp.float32),
    ],
)
def _sc_counts(dst_hbm, et_hbm, zeros_hbm, out_hbm, oh_v, dst_v, et_v, shared):
    c = lax.axis_index("c")
    s = lax.axis_index("s")

    @pl.when(c == 0)
    def _():
        _striped(lambda off, n: pltpu.sync_copy(
            zeros_hbm.at[pl.ds(off, n)], shared.at[pl.ds(off, n)]))
        base = pl.multiple_of(s * EPW1, EPW1)
        pltpu.sync_copy(
            dst_hbm.at[pl.ds(pl.multiple_of(base // 128, 8), EPW1 // 128)], dst_v)
        pltpu.sync_copy(et_hbm.at[pl.ds(base, EPW1)], et_v)
        iota16 = lax.iota(jnp.int32, 16)
        ones16 = jnp.ones((16,), jnp.float32)
        zeros16 = jnp.zeros((16,), jnp.float32)

        def zrow(i, cc):
            oh_v[i, :] = zeros16
            return cc

        lax.fori_loop(0, CCHUNK, zrow, 0)
        plsc.subcore_barrier()

        def chunk(k, carry):
            def mark(g, cc):
                i_vec = g * 16 + iota16
                et16 = et_v[pl.ds(k * CCHUNK + g * 16, 16)]
                mask = base + k * CCHUNK + i_vec < NEDGE
                plsc.store_scatter(oh_v, [i_vec, et16], ones16, mask=mask)
                return cc

            lax.fori_loop(0, CCHUNK // 16, mark, 0)
            for j in range(CCHUNK // 128):
                pltpu.sync_copy(
                    oh_v.at[pl.ds(j * 128, 128)],
                    shared.at[dst_v.at[pl.multiple_of(k * (CCHUNK // 128), 8) + j]],
                    add=True)

            def unmark(g, cc):
                i_vec = g * 16 + iota16
                et16 = et_v[pl.ds(k * CCHUNK + g * 16, 16)]
                plsc.store_scatter(oh_v, [i_vec, et16], zeros16)
                return cc

            lax.fori_loop(0, CCHUNK // 16, unmark, 0)
            return carry

        lax.fori_loop(0, EPW1 // CCHUNK, chunk, 0)
        plsc.subcore_barrier()
        _striped(lambda off, n: pltpu.sync_copy(
            shared.at[pl.ds(off, n)], out_hbm.at[pl.ds(off, n)]))


@functools.partial(
    pl.kernel,
    out_type=jax.ShapeDtypeStruct((EPAD,), jnp.float32),
    mesh=_mesh,
    compiler_params=_sc_params,
    scratch_types=[
        pltpu.VMEM((EPW // 128, 128), jnp.int32),
        pltpu.VMEM((EPW,), jnp.int32),
        pltpu.VMEM((EPW, NREL), jnp.float32),
        pltpu.VMEM((EPW,), jnp.float32),
        pltpu.SemaphoreType.DMA,
    ],
)
def _sc_norm(c2d_hbm, dst_hbm, et_hbm, out_hbm, dst_v, et_v, crows_v, norm_v, sem):
    w = _wid()
    base = pl.multiple_of(w * EPW, EPW)
    pltpu.sync_copy(
        dst_hbm.at[pl.ds(pl.multiple_of(base // 128, 8), EPW // 128)], dst_v)
    pltpu.sync_copy(et_hbm.at[pl.ds(base, EPW)], et_v)
    descs = [
        pltpu.async_copy(
            c2d_hbm.at[dst_v.at[j]], crows_v.at[pl.ds(j * 128, 128)], sem)
        for j in range(EPW // 128)
    ]
    for d in descs:
        d.wait()
    iota16 = lax.iota(jnp.int32, 16)
    one = jnp.ones((16,), jnp.float32)

    def grp(g, carry):
        i_vec = g * 16 + iota16
        et16 = et_v[pl.ds(g * 16, 16)]
        cv = plsc.load_gather(crows_v, [i_vec, et16])
        inv = one / jnp.maximum(cv, one)
        inv = jnp.where(base + i_vec < NEDGE, inv, jnp.zeros((16,), jnp.float32))
        norm_v[pl.ds(g * 16, 16)] = inv
        return carry

    lax.fori_loop(0, EPW // 16, grp, 0)
    pltpu.sync_copy(norm_v, out_hbm.at[pl.ds(base, EPW)])


MSEG = 128
HALF = 512
SEGH = HALF // MSEG
NH = EPW // HALF


@functools.partial(
    pl.kernel,
    out_type=jax.ShapeDtypeStruct((NCORE, NNODES, HID), jnp.float32),
    mesh=_mesh,
    compiler_params=_sc_params,
    scratch_types=[
        pltpu.VMEM((EPW // 128, 128), jnp.int32),
        pltpu.VMEM((EPW // 128, 128), jnp.int32),
        pltpu.VMEM((EPW,), jnp.float32),
        pltpu.VMEM((HALF, HID), jnp.float32),
        pltpu.VMEM((HALF, HID), jnp.float32),
        pltpu.VMEM_SHARED((NNODES, HID), jnp.float32),
        pltpu.SemaphoreType.DMA,
        pltpu.SemaphoreType.DMA,
    ],
)
def _sc_messages(ytab_hbm, gidx_hbm, dst_hbm, norm_hbm, zeros_hbm, out_hbm,
                 gidx_v, dst_v, norm_v, rows_a, rows_b, shared, sem_a, sem_b):
    c = lax.axis_index("c")
    w = _wid()
    _striped(lambda off, n: pltpu.sync_copy(
        zeros_hbm.at[pl.ds(off, n)], shared.at[pl.ds(off, n)]))
    base = pl.multiple_of(w * EPW, EPW)
    rbase = pl.multiple_of(base // 128, 8)
    pltpu.sync_copy(gidx_hbm.at[pl.ds(rbase, EPW // 128)], gidx_v)
    pltpu.sync_copy(dst_hbm.at[pl.ds(rbase, EPW // 128)], dst_v)
    pltpu.sync_copy(norm_hbm.at[pl.ds(base, EPW)], norm_v)
    plsc.subcore_barrier()

    bufs = (rows_a, rows_b)
    sems = (sem_a, sem_b)

    def fire(t):
        buf = bufs[t % 2]
        sem = sems[t % 2]
        return [
            pltpu.async_copy(
                ytab_hbm.at[gidx_v.at[t * SEGH + j]],
                buf.at[pl.ds(j * MSEG, MSEG)], sem)
            for j in range(SEGH)
        ]

    pending = fire(0)
    for t in range(NH):
        for d in pending:
            d.wait()
        if t + 1 < NH:
            pending = fire(t + 1)
        buf = bufs[t % 2]

        def grp(g, cc):
            nv16 = norm_v[pl.ds(t * HALF + g * 16, 16)]
            i0 = g * 16
            for l in range(16):
                nv = jnp.broadcast_to(nv16[l], (16,))
                for q in range(HID // 16):
                    sl = pl.ds(q * 16, 16)
                    buf[i0 + l, sl] = buf[i0 + l, sl] * nv
            return cc

        lax.fori_loop(0, HALF // 16, grp, 0)
        for j in range(SEGH):
            pltpu.sync_copy(
                buf.at[pl.ds(j * MSEG, MSEG)],
                shared.at[dst_v.at[t * SEGH + j]], add=True)
    plsc.subcore_barrier()
    _striped(lambda off, n: pltpu.sync_copy(
        shared.at[pl.ds(off, n)], out_hbm.at[c].at[pl.ds(off, n)]))


@functools.partial(
    pl.kernel,
    out_type=jax.ShapeDtypeStruct((3, 2 * NTGT, HID), jnp.float32),
    mesh=_mesh,
    compiler_params=_sc_params,
    scratch_types=[
        pltpu.VMEM((TPW,), jnp.int32),
        pltpu.VMEM((TPW, HID), jnp.float32),
        pltpu.SemaphoreType.DMA,
    ],
)
def _sc_gather_targets(ap_hbm, hr_hbm, idx_hbm, out_hbm, idx_v, rows_v, sem):
    w = _wid()
    base = pl.multiple_of(w * TPW, TPW)
    pltpu.sync_copy(idx_hbm.at[pl.ds(base, TPW)], idx_v)
    for c in range(NCORE):
        pltpu.async_copy(ap_hbm.at[c].at[idx_v], rows_v, sem).wait()
        pltpu.sync_copy(rows_v, out_hbm.at[c].at[pl.ds(base, TPW)])
    pltpu.async_copy(hr_hbm.at[idx_v], rows_v, sem).wait()
    pltpu.sync_copy(rows_v, out_hbm.at[2].at[pl.ds(base, TPW)])


def _mm_body(a_ref, w_ref, y_ref, r_ref):
    p = jnp.dot(a_ref[...], w_ref[...], preferred_element_type=jnp.float32)
    r_ref[...] = p[:, :HID]
    for r in range(NREL):
        y_ref[r] = p[:, 2 * HID + r * HID:2 * HID + (r + 1) * HID]


def _tc_matmul(h, wcat):
    m, kdim = h.shape
    ncols = wcat.shape[1]
    bm = 400
    return pl.pallas_call(
        _mm_body,
        grid=(m // bm,),
        in_specs=[
            pl.BlockSpec((bm, kdim), lambda i: (i, 0)),
            pl.BlockSpec((kdim, ncols), lambda i: (0, 0)),
        ],
        out_specs=[
            pl.BlockSpec((NREL, bm, HID), lambda i: (0, i, 0)),
            pl.BlockSpec((bm, HID), lambda i: (i, 0)),
        ],
        out_shape=[
            jax.ShapeDtypeStruct((NREL, m, HID), jnp.float32),
            jax.ShapeDtypeStruct((m, HID), jnp.float32),
        ],
    )(h, wcat)


def _mm_fused_body(p0_ref, p1_ref, hr_ref, b_ref, w_ref, y_ref, r_ref):
    a = jnp.maximum(p0_ref[...] + p1_ref[...] + hr_ref[...] + b_ref[...], 0.0)
    p = jnp.dot(a, w_ref[...], preferred_element_type=jnp.float32)
    r_ref[...] = p[:, :HID]
    for r in range(NREL):
        y_ref[r] = p[:, 2 * HID + r * HID:2 * HID + (r + 1) * HID]


def _tc_matmul_fused(p0, p1, hr, bias, wcat):
    m = p0.shape[0]
    ncols = wcat.shape[1]
    bm = 400
    spec = pl.BlockSpec((bm, HID), lambda i: (i, 0))
    return pl.pallas_call(
        _mm_fused_body,
        grid=(m // bm,),
        in_specs=[
            spec, spec, spec,
            pl.BlockSpec((1, HID), lambda i: (0, 0)),
            pl.BlockSpec((HID, ncols), lambda i: (0, 0)),
        ],
        out_specs=[
            pl.BlockSpec((NREL, bm, HID), lambda i: (0, i, 0)),
            pl.BlockSpec((bm, HID), lambda i: (i, 0)),
        ],
        out_shape=[
            jax.ShapeDtypeStruct((NREL, m, HID), jnp.float32),
            jax.ShapeDtypeStruct((m, HID), jnp.float32),
        ],
    )(p0, p1, hr, bias.reshape(1, HID), wcat)


def _dec_body(g_ref, b_ref, w2_ref, s_ref, o_ref):
    h = jnp.maximum(g_ref[0] + g_ref[1] + g_ref[2] + b_ref[...], 0.0)
    x1 = h[:NTGT]
    x2 = h[NTGT:]
    p = jnp.dot(x1, w2_ref[...], preferred_element_type=jnp.float32)
    x2t = jnp.concatenate([x2] * NREL, axis=1)
    rec = jnp.dot(p * x2t, s_ref[...], preferred_element_type=jnp.float32)
    o_ref[...] = jax.nn.sigmoid(rec)


def _tc_decoder(g3, bias, w2cat, sel):
    return pl.pallas_call(
        _dec_body,
        out_shape=jax.ShapeDtypeStruct((NTGT, NREL), jnp.float32),
    )(g3, bias.reshape(1, HID), w2cat, sel)


def kernel(x, edge_index, edge_attr, target_edge_index,
           bases0, comp0, root0, bias0, bases1, comp1, root1, bias1, R, D):
    f32 = jnp.float32
    et = jnp.argmax(edge_attr, axis=1).astype(jnp.int32)
    src = edge_index[0]
    dst = edge_index[1]
    gidx = et * NNODES + src
    pad = EPAD - NEDGE
    gidx_p = jnp.pad(gidx, (0, pad)).reshape(EPAD // 128, 128)
    dst_p = jnp.pad(dst, (0, pad)).reshape(EPAD // 128, 128)
    et_p = jnp.pad(et, (0, pad))
    zeros_n16 = jnp.zeros((NNODES, NREL), f32)
    zeros_nh = jnp.zeros((NNODES, HID), f32)

    c2d = _sc_counts(dst_p, et_p, zeros_n16)
    norm = _sc_norm(c2d, dst_p, et_p)

    def wcat(comp, bases, root):
        indim = root.shape[0]
        w = jnp.einsum('rb,bio->rio', comp, bases)
        w = w.transpose(1, 0, 2).reshape(indim, NREL * HID)
        return jnp.concatenate([root, jnp.zeros((indim, HID), f32), w], axis=1)

    y0, xr0 = _tc_matmul(x, wcat(comp0, bases0, root0))
    ap0 = _sc_messages(y0.reshape(NREL * NNODES, HID), gidx_p, dst_p, norm,
                       zeros_nh)

    y1, hr1 = _tc_matmul_fused(ap0[0], ap0[1], xr0, bias0,
                               wcat(comp1, bases1, root1))
    ap1 = _sc_messages(y1.reshape(NREL * NNODES, HID), gidx_p, dst_p, norm,
                       zeros_nh)
    tgt = target_edge_index.reshape(-1).astype(jnp.int32)
    g3 = _sc_gather_targets(ap1, hr1, tgt)
    w2 = (D[:, :, None] * R[None, :, :] * D[:, None, :])
    w2 = w2.transpose(1, 0, 2).reshape(HID, NREL * HID)
    sel = (lax.broadcasted_iota(jnp.int32, (NREL * HID, NREL), 0) // HID ==
           lax.broadcasted_iota(jnp.int32, (NREL * HID, NREL), 1)).astype(f32)
    return _tc_decoder(g3, bias1, w2, sel)


# speedup vs baseline: 1.0786x; 1.0008x over previous
import functools

import jax
import jax.numpy as jnp
from jax import lax
from jax.experimental import pallas as pl
from jax.experimental.pallas import tpu as pltpu
from jax.experimental.pallas import tpu_sc as plsc

NNODES = 10000
NREL = 16
HID = 64
NEDGE = 160000
NTGT = 1024
NCORE = 2
NSUB = 16
NWORK = NCORE * NSUB
EPAD = 163840
EPW = EPAD // NWORK
EPW1 = EPAD // NSUB
MCHUNK = 1024
CCHUNK = 1024
NSTRIPE = 624
NSTRIPE_LAST = NNODES - (NSUB - 1) * NSTRIPE
TPW = 2 * NTGT // NWORK

_mesh = plsc.VectorSubcoreMesh(
    core_axis_name="c", subcore_axis_name="s", num_cores=NCORE, num_subcores=NSUB
)
_sc_params = pltpu.CompilerParams(use_tc_tiling_on_sc=False, needs_layout_passes=False)


def _wid():
    return lax.axis_index("s") * NCORE + lax.axis_index("c")


def _striped(fn):
    s = lax.axis_index("s")

    @pl.when(s < NSUB - 1)
    def _():
        fn(pl.multiple_of(s * NSTRIPE, 8), NSTRIPE)

    @pl.when(s == NSUB - 1)
    def _():
        fn((NSUB - 1) * NSTRIPE, NSTRIPE_LAST)


@functools.partial(
    pl.kernel,
    out_type=jax.ShapeDtypeStruct((NNODES, NREL), jnp.float32),
    mesh=_mesh,
    compiler_params=_sc_params,
    scratch_types=[
        pltpu.VMEM((CCHUNK, NREL), jnp.float32),
        pltpu.VMEM((EPW1 // 128, 128), jnp.int32),
        pltpu.VMEM((EPW1,), jnp.int32),
        pltpu.VMEM_SHARED((NNODES, NREL), jnp.float32),
    ],
)
def _sc_counts(dst_hbm, et_hbm, zeros_hbm, out_hbm, oh_v, dst_v, et_v, shared):
    c = lax.axis_index("c")
    s = lax.axis_index("s")

    @pl.when(c == 0)
    def _():
        _striped(lambda off, n: pltpu.sync_copy(
            zeros_hbm.at[pl.ds(off, n)], shared.at[pl.ds(off, n)]))
        base = pl.multiple_of(s * EPW1, EPW1)
        pltpu.sync_copy(
            dst_hbm.at[pl.ds(pl.multiple_of(base // 128, 8), EPW1 // 128)], dst_v)
        pltpu.sync_copy(et_hbm.at[pl.ds(base, EPW1)], et_v)
        iota16 = lax.iota(jnp.int32, 16)
        ones16 = jnp.ones((16,), jnp.float32)
        zeros16 = jnp.zeros((16,), jnp.float32)

        def zrow(i, cc):
            oh_v[i, :] = zeros16
            return cc

        lax.fori_loop(0, CCHUNK, zrow, 0)
        plsc.subcore_barrier()

        def chunk(k, carry):
            def mark(g, cc):
                i_vec = g * 16 + iota16
                et16 = et_v[pl.ds(k * CCHUNK + g * 16, 16)]
                mask = base + k * CCHUNK + i_vec < NEDGE
                plsc.store_scatter(oh_v, [i_vec, et16], ones16, mask=mask)
                return cc

            lax.fori_loop(0, CCHUNK // 16, mark, 0)
            for j in range(CCHUNK // 128):
                pltpu.sync_copy(
                    oh_v.at[pl.ds(j * 128, 128)],
                    shared.at[dst_v.at[pl.multiple_of(k * (CCHUNK // 128), 8) + j]],
                    add=True)

            def unmark(g, cc):
                i_vec = g * 16 + iota16
                et16 = et_v[pl.ds(k * CCHUNK + g * 16, 16)]
                plsc.store_scatter(oh_v, [i_vec, et16], zeros16)
                return cc

            lax.fori_loop(0, CCHUNK // 16, unmark, 0)
            return carry

        lax.fori_loop(0, EPW1 // CCHUNK, chunk, 0)
        plsc.subcore_barrier()
        _striped(lambda off, n: pltpu.sync_copy(
            shared.at[pl.ds(off, n)], out_hbm.at[pl.ds(off, n)]))


@functools.partial(
    pl.kernel,
    out_type=jax.ShapeDtypeStruct((EPAD,), jnp.float32),
    mesh=_mesh,
    compiler_params=_sc_params,
    scratch_types=[
        pltpu.VMEM((EPW // 128, 128), jnp.int32),
        pltpu.VMEM((EPW,), jnp.int32),
        pltpu.VMEM((EPW, NREL), jnp.float32),
        pltpu.VMEM((EPW,), jnp.float32),
        pltpu.SemaphoreType.DMA,
    ],
)
def _sc_norm(c2d_hbm, dst_hbm, et_hbm, out_hbm, dst_v, et_v, crows_v, norm_v, sem):
    w = _wid()
    base = pl.multiple_of(w * EPW, EPW)
    pltpu.sync_copy(
        dst_hbm.at[pl.ds(pl.multiple_of(base // 128, 8), EPW // 128)], dst_v)
    pltpu.sync_copy(et_hbm.at[pl.ds(base, EPW)], et_v)
    descs = [
        pltpu.async_copy(
            c2d_hbm.at[dst_v.at[j]], crows_v.at[pl.ds(j * 128, 128)], sem)
        for j in range(EPW // 128)
    ]
    for d in descs:
        d.wait()
    iota16 = lax.iota(jnp.int32, 16)
    one = jnp.ones((16,), jnp.float32)

    def grp(g, carry):
        i_vec = g * 16 + iota16
        et16 = et_v[pl.ds(g * 16, 16)]
        cv = plsc.load_gather(crows_v, [i_vec, et16])
        inv = one / jnp.maximum(cv, one)
        inv = jnp.where(base + i_vec < NEDGE, inv, jnp.zeros((16,), jnp.float32))
        norm_v[pl.ds(g * 16, 16)] = inv
        return carry

    lax.fori_loop(0, EPW // 16, grp, 0)
    pltpu.sync_copy(norm_v, out_hbm.at[pl.ds(base, EPW)])


MSEG = 128
HALF = 512
SEGH = HALF // MSEG
NH = EPW // HALF


def _msg_pipeline(ytab_hbm, gidx_v, dst_v, norm_v, rows_a, rows_b,
                  shared, sem_a, sem_b):
    bufs = (rows_a, rows_b)
    sems = (sem_a, sem_b)

    def fire(t):
        buf = bufs[t % 2]
        sem = sems[t % 2]
        return [
            pltpu.async_copy(
                ytab_hbm.at[gidx_v.at[t * SEGH + j]],
                buf.at[pl.ds(j * MSEG, MSEG)], sem)
            for j in range(SEGH)
        ]

    pending = fire(0)
    for t in range(NH):
        for d in pending:
            d.wait()
        if t + 1 < NH:
            pending = fire(t + 1)
        buf = bufs[t % 2]

        def grp(g, cc):
            nv16 = norm_v[pl.ds(t * HALF + g * 16, 16)]
            i0 = g * 16
            for l in range(16):
                nv = jnp.broadcast_to(nv16[l], (16,))
                for q in range(HID // 16):
                    sl = pl.ds(q * 16, 16)
                    buf[i0 + l, sl] = buf[i0 + l, sl] * nv
            return cc

        lax.fori_loop(0, HALF // 16, grp, 0)
        for j in range(SEGH):
            pltpu.sync_copy(
                buf.at[pl.ds(j * MSEG, MSEG)],
                shared.at[dst_v.at[t * SEGH + j]], add=True)


@functools.partial(
    pl.kernel,
    out_type=jax.ShapeDtypeStruct((NCORE, NNODES, HID), jnp.float32),
    mesh=_mesh,
    compiler_params=_sc_params,
    scratch_types=[
        pltpu.VMEM((EPW // 128, 128), jnp.int32),
        pltpu.VMEM((EPW // 128, 128), jnp.int32),
        pltpu.VMEM((EPW,), jnp.float32),
        pltpu.VMEM((HALF, HID), jnp.float32),
        pltpu.VMEM((HALF, HID), jnp.float32),
        pltpu.VMEM_SHARED((NNODES, HID), jnp.float32),
        pltpu.SemaphoreType.DMA,
        pltpu.SemaphoreType.DMA,
    ],
)
def _sc_messages(ytab_hbm, gidx_hbm, dst_hbm, norm_hbm, zeros_hbm, out_hbm,
                 gidx_v, dst_v, norm_v, rows_a, rows_b, shared, sem_a, sem_b):
    c = lax.axis_index("c")
    w = _wid()
    _striped(lambda off, n: pltpu.sync_copy(
        zeros_hbm.at[pl.ds(off, n)], shared.at[pl.ds(off, n)]))
    base = pl.multiple_of(w * EPW, EPW)
    rbase = pl.multiple_of(base // 128, 8)
    pltpu.sync_copy(gidx_hbm.at[pl.ds(rbase, EPW // 128)], gidx_v)
    pltpu.sync_copy(dst_hbm.at[pl.ds(rbase, EPW // 128)], dst_v)
    pltpu.sync_copy(norm_hbm.at[pl.ds(base, EPW)], norm_v)
    plsc.subcore_barrier()
    _msg_pipeline(ytab_hbm, gidx_v, dst_v, norm_v, rows_a, rows_b,
                  shared, sem_a, sem_b)
    plsc.subcore_barrier()
    _striped(lambda off, n: pltpu.sync_copy(
        shared.at[pl.ds(off, n)], out_hbm.at[c].at[pl.ds(off, n)]))


@functools.partial(
    pl.kernel,
    out_type=[
        jax.ShapeDtypeStruct((NCORE, NNODES, HID), jnp.float32),
        jax.ShapeDtypeStruct((3, 2 * NTGT, HID), jnp.float32),
    ],
    mesh=_mesh,
    compiler_params=_sc_params,
    scratch_types=[
        pltpu.VMEM((EPW // 128, 128), jnp.int32),
        pltpu.VMEM((EPW // 128, 128), jnp.int32),
        pltpu.VMEM((EPW,), jnp.float32),
        pltpu.VMEM((HALF, HID), jnp.float32),
        pltpu.VMEM((HALF, HID), jnp.float32),
        pltpu.VMEM_SHARED((NNODES, HID), jnp.float32),
        pltpu.SemaphoreType.DMA,
        pltpu.SemaphoreType.DMA,
        pltpu.VMEM((128,), jnp.int32),
        pltpu.VMEM((TPW,), jnp.int32),
    ],
)
def _sc_messages_tgt(ytab_hbm, gidx_hbm, dst_hbm, norm_hbm, hr_hbm, tgt_hbm,
                     zeros_hbm, out_hbm, g3_hbm,
                     gidx_v, dst_v, norm_v, rows_a, rows_b, shared,
                     sem_a, sem_b, idxc_v, idxh_v):
    c = lax.axis_index("c")
    s = lax.axis_index("s")
    w = _wid()
    _striped(lambda off, n: pltpu.sync_copy(
        zeros_hbm.at[pl.ds(off, n)], shared.at[pl.ds(off, n)]))
    base = pl.multiple_of(w * EPW, EPW)
    rbase = pl.multiple_of(base // 128, 8)
    pltpu.sync_copy(gidx_hbm.at[pl.ds(rbase, EPW // 128)], gidx_v)
    pltpu.sync_copy(dst_hbm.at[pl.ds(rbase, EPW // 128)], dst_v)
    pltpu.sync_copy(norm_hbm.at[pl.ds(base, EPW)], norm_v)
    plsc.subcore_barrier()
    _msg_pipeline(ytab_hbm, gidx_v, dst_v, norm_v, rows_a, rows_b,
                  shared, sem_a, sem_b)
    plsc.subcore_barrier()
    _striped(lambda off, n: pltpu.sync_copy(
        shared.at[pl.ds(off, n)], out_hbm.at[c].at[pl.ds(off, n)]))
    plsc.subcore_barrier()
    tb = pl.multiple_of(s * 128, 8)
    pltpu.sync_copy(tgt_hbm.at[pl.ds(tb, 128)], idxc_v)
    pltpu.async_copy(
        out_hbm.at[c].at[idxc_v], rows_a.at[pl.ds(0, 128)], sem_a).wait()
    pltpu.sync_copy(rows_a.at[pl.ds(0, 128)], g3_hbm.at[c].at[pl.ds(tb, 128)])
    hb = pl.multiple_of(w * TPW, 8)
    pltpu.sync_copy(tgt_hbm.at[pl.ds(hb, TPW)], idxh_v)
    pltpu.async_copy(hr_hbm.at[idxh_v], rows_b.at[pl.ds(0, TPW)], sem_b).wait()
    pltpu.sync_copy(rows_b.at[pl.ds(0, TPW)], g3_hbm.at[2].at[pl.ds(hb, TPW)])


def _mm_body(a_ref, w_ref, y_ref, r_ref):
    p = jnp.dot(a_ref[...], w_ref[...], preferred_element_type=jnp.float32)
    r_ref[...] = p[:, :HID]
    for r in range(NREL):
        y_ref[r] = p[:, 2 * HID + r * HID:2 * HID + (r + 1) * HID]


def _tc_matmul(h, wcat):
    m, kdim = h.shape
    ncols = wcat.shape[1]
    bm = 400
    return pl.pallas_call(
        _mm_body,
        grid=(m // bm,),
        in_specs=[
            pl.BlockSpec((bm, kdim), lambda i: (i, 0)),
            pl.BlockSpec((kdim, ncols), lambda i: (0, 0)),
        ],
        out_specs=[
            pl.BlockSpec((NREL, bm, HID), lambda i: (0, i, 0)),
            pl.BlockSpec((bm, HID), lambda i: (i, 0)),
        ],
        out_shape=[
            jax.ShapeDtypeStruct((NREL, m, HID), jnp.float32),
            jax.ShapeDtypeStruct((m, HID), jnp.float32),
        ],
    )(h, wcat)


def _mm_fused_body(p0_ref, p1_ref, hr_ref, b_ref, w_ref, y_ref, r_ref):
    a = jnp.maximum(p0_ref[...] + p1_ref[...] + hr_ref[...] + b_ref[...], 0.0)
    p = jnp.dot(a, w_ref[...], preferred_element_type=jnp.float32)
    r_ref[...] = p[:, :HID]
    for r in range(NREL):
        y_ref[r] = p[:, 2 * HID + r * HID:2 * HID + (r + 1) * HID]


def _tc_matmul_fused(p0, p1, hr, bias, wcat):
    m = p0.shape[0]
    ncols = wcat.shape[1]
    bm = 400
    spec = pl.BlockSpec((bm, HID), lambda i: (i, 0))
    return pl.pallas_call(
        _mm_fused_body,
        grid=(m // bm,),
        in_specs=[
            spec, spec, spec,
            pl.BlockSpec((1, HID), lambda i: (0, 0)),
            pl.BlockSpec((HID, ncols), lambda i: (0, 0)),
        ],
        out_specs=[
            pl.BlockSpec((NREL, bm, HID), lambda i: (0, i, 0)),
            pl.BlockSpec((bm, HID), lambda i: (i, 0)),
        ],
        out_shape=[
            jax.ShapeDtypeStruct((NREL, m, HID), jnp.float32),
            jax.ShapeDtypeStruct((m, HID), jnp.float32),
        ],
    )(p0, p1, hr, bias.reshape(1, HID), wcat)


def _dec_body(g_ref, b_ref, w2_ref, s_ref, o_ref):
    h = jnp.maximum(g_ref[0] + g_ref[1] + g_ref[2] + b_ref[...], 0.0)
    x1 = h[:NTGT]
    x2 = h[NTGT:]
    p = jnp.dot(x1, w2_ref[...], preferred_element_type=jnp.float32)
    x2t = jnp.concatenate([x2] * NREL, axis=1)
    rec = jnp.dot(p * x2t, s_ref[...], preferred_element_type=jnp.float32)
    o_ref[...] = jax.nn.sigmoid(rec)


def _tc_decoder(g3, bias, w2cat, sel):
    return pl.pallas_call(
        _dec_body,
        out_shape=jax.ShapeDtypeStruct((NTGT, NREL), jnp.float32),
    )(g3, bias.reshape(1, HID), w2cat, sel)


def kernel(x, edge_index, edge_attr, target_edge_index,
           bases0, comp0, root0, bias0, bases1, comp1, root1, bias1, R, D):
    f32 = jnp.float32
    et = jnp.argmax(edge_attr, axis=1).astype(jnp.int32)
    src = edge_index[0]
    dst = edge_index[1]
    gidx = et * NNODES + src
    pad = EPAD - NEDGE
    gidx_p = jnp.pad(gidx, (0, pad)).reshape(EPAD // 128, 128)
    dst_p = jnp.pad(dst, (0, pad)).reshape(EPAD // 128, 128)
    et_p = jnp.pad(et, (0, pad))
    zeros_n16 = jnp.zeros((NNODES, NREL), f32)
    zeros_nh = jnp.zeros((NNODES, HID), f32)

    c2d = _sc_counts(dst_p, et_p, zeros_n16)
    norm = _sc_norm(c2d, dst_p, et_p)

    def wcat(comp, bases, root):
        indim = root.shape[0]
        w = jnp.einsum('rb,bio->rio', comp, bases)
        w = w.transpose(1, 0, 2).reshape(indim, NREL * HID)
        return jnp.concatenate([root, jnp.zeros((indim, HID), f32), w], axis=1)

    y0, xr0 = _tc_matmul(x, wcat(comp0, bases0, root0))
    ap0 = _sc_messages(y0.reshape(NREL * NNODES, HID), gidx_p, dst_p, norm,
                       zeros_nh)

    y1, hr1 = _tc_matmul_fused(ap0[0], ap0[1], xr0, bias0,
                               wcat(comp1, bases1, root1))
    tgt = target_edge_index.reshape(-1).astype(jnp.int32)
    _, g3 = _sc_messages_tgt(y1.reshape(NREL * NNODES, HID), gidx_p, dst_p,
                             norm, hr1, tgt, zeros_nh)
    w2 = (D[:, :, None] * R[None, :, :] * D[:, None, :])
    w2 = w2.transpose(1, 0, 2).reshape(HID, NREL * HID)
    sel = (lax.broadcasted_iota(jnp.int32, (NREL * HID, NREL), 0) // HID ==
           lax.broadcasted_iota(jnp.int32, (NREL * HID, NREL), 1)).astype(f32)
    return _tc_decoder(g3, bias1, w2, sel)


# speedup vs baseline: 1.0792x; 1.0006x over previous
import functools

import jax
import jax.numpy as jnp
from jax import lax
from jax.experimental import pallas as pl
from jax.experimental.pallas import tpu as pltpu
from jax.experimental.pallas import tpu_sc as plsc

NNODES = 10000
NREL = 16
HID = 64
NEDGE = 160000
NTGT = 1024
NCORE = 2
NSUB = 16
NWORK = NCORE * NSUB
EPAD = 163840
EPW = EPAD // NWORK
EPW1 = EPAD // NSUB
CCHUNK = 1024
NSTRIPE = 624
NSTRIPE_LAST = NNODES - (NSUB - 1) * NSTRIPE
TPW = 2 * NTGT // NWORK

_mesh = plsc.VectorSubcoreMesh(
    core_axis_name="c", subcore_axis_name="s", num_cores=NCORE, num_subcores=NSUB
)
_sc_params = pltpu.CompilerParams(use_tc_tiling_on_sc=False, needs_layout_passes=False)


def _wid():
    return lax.axis_index("s") * NCORE + lax.axis_index("c")


def _striped(fn):
    s = lax.axis_index("s")

    @pl.when(s < NSUB - 1)
    def _():
        fn(pl.multiple_of(s * NSTRIPE, 8), NSTRIPE)

    @pl.when(s == NSUB - 1)
    def _():
        fn((NSUB - 1) * NSTRIPE, NSTRIPE_LAST)


@functools.partial(
    pl.kernel,
    out_type=jax.ShapeDtypeStruct((NNODES, NREL), jnp.float32),
    mesh=_mesh,
    compiler_params=_sc_params,
    scratch_types=[
        pltpu.VMEM((CCHUNK, NREL), jnp.float32),
        pltpu.VMEM((EPW1 // 128, 128), jnp.int32),
        pltpu.VMEM((EPW1,), jnp.int32),
        pltpu.VMEM_SHARED((NNODES, NREL), jnp.float32),
    ],
)
def _sc_counts(dst_hbm, et_hbm, zeros_hbm, out_hbm, oh_v, dst_v, et_v, shared):
    c = lax.axis_index("c")
    s = lax.axis_index("s")

    @pl.when(c == 0)
    def _():
        _striped(lambda off, n: pltpu.sync_copy(
            zeros_hbm.at[pl.ds(off, n)], shared.at[pl.ds(off, n)]))
        base = pl.multiple_of(s * EPW1, EPW1)
        pltpu.sync_copy(
            dst_hbm.at[pl.ds(pl.multiple_of(base // 128, 8), EPW1 // 128)], dst_v)
        pltpu.sync_copy(et_hbm.at[pl.ds(base, EPW1)], et_v)
        iota16 = lax.iota(jnp.int32, 16)
        ones16 = jnp.ones((16,), jnp.float32)
        zeros16 = jnp.zeros((16,), jnp.float32)

        def zrow(i, cc):
            oh_v[i, :] = zeros16
            return cc

        lax.fori_loop(0, CCHUNK, zrow, 0)
        plsc.subcore_barrier()

        def chunk(k, carry):
            def mark(g, cc):
                i_vec = g * 16 + iota16
                et16 = et_v[pl.ds(k * CCHUNK + g * 16, 16)]
                mask = base + k * CCHUNK + i_vec < NEDGE
                plsc.store_scatter(oh_v, [i_vec, et16], ones16, mask=mask)
                return cc

            lax.fori_loop(0, CCHUNK // 16, mark, 0)
            for j in range(CCHUNK // 128):
                pltpu.sync_copy(
                    oh_v.at[pl.ds(j * 128, 128)],
                    shared.at[dst_v.at[pl.multiple_of(k * (CCHUNK // 128), 8) + j]],
                    add=True)

            def unmark(g, cc):
                i_vec = g * 16 + iota16
                et16 = et_v[pl.ds(k * CCHUNK + g * 16, 16)]
                plsc.store_scatter(oh_v, [i_vec, et16], zeros16)
                return cc

            lax.fori_loop(0, CCHUNK // 16, unmark, 0)
            return carry

        lax.fori_loop(0, EPW1 // CCHUNK, chunk, 0)
        plsc.subcore_barrier()
        _striped(lambda off, n: pltpu.sync_copy(
            shared.at[pl.ds(off, n)], out_hbm.at[pl.ds(off, n)]))


@functools.partial(
    pl.kernel,
    out_type=jax.ShapeDtypeStruct((EPAD,), jnp.float32),
    mesh=_mesh,
    compiler_params=_sc_params,
    scratch_types=[
        pltpu.VMEM((EPW // 128, 128), jnp.int32),
        pltpu.VMEM((EPW,), jnp.int32),
        pltpu.VMEM((EPW, NREL), jnp.float32),
        pltpu.VMEM((EPW,), jnp.float32),
        pltpu.SemaphoreType.DMA,
    ],
)
def _sc_norm(c2d_hbm, dst_hbm, et_hbm, out_hbm, dst_v, et_v, crows_v, norm_v, sem):
    w = _wid()
    base = pl.multiple_of(w * EPW, EPW)
    pltpu.sync_copy(
        dst_hbm.at[pl.ds(pl.multiple_of(base // 128, 8), EPW // 128)], dst_v)
    pltpu.sync_copy(et_hbm.at[pl.ds(base, EPW)], et_v)
    descs = [
        pltpu.async_copy(
            c2d_hbm.at[dst_v.at[j]], crows_v.at[pl.ds(j * 128, 128)], sem)
        for j in range(EPW // 128)
    ]
    for d in descs:
        d.wait()
    iota16 = lax.iota(jnp.int32, 16)
    one = jnp.ones((16,), jnp.float32)

    def grp(g, carry):
        i_vec = g * 16 + iota16
        et16 = et_v[pl.ds(g * 16, 16)]
        cv = plsc.load_gather(crows_v, [i_vec, et16])
        inv = one / jnp.maximum(cv, one)
        inv = jnp.where(base + i_vec < NEDGE, inv, jnp.zeros((16,), jnp.float32))
        norm_v[pl.ds(g * 16, 16)] = inv
        return carry

    lax.fori_loop(0, EPW // 16, grp, 0)
    pltpu.sync_copy(norm_v, out_hbm.at[pl.ds(base, EPW)])


MSEG = 128
HALF = 512
SEGH = HALF // MSEG
NH = EPW // HALF


def _msg_pipeline(ytab_hbm, gidx_v, dst_v, norm_v, rows_a, rows_b,
                  shared, sem_a, sem_b):
    bufs = (rows_a, rows_b)
    sems = (sem_a, sem_b)

    def fire(t):
        buf = bufs[t % 2]
        sem = sems[t % 2]
        return [
            pltpu.async_copy(
                ytab_hbm.at[gidx_v.at[t * SEGH + j]],
                buf.at[pl.ds(j * MSEG, MSEG)], sem)
            for j in range(SEGH)
        ]

    pending = fire(0)
    for t in range(NH):
        for d in pending:
            d.wait()
        if t + 1 < NH:
            pending = fire(t + 1)
        buf = bufs[t % 2]

        def grp(g, cc):
            nv16 = norm_v[pl.ds(t * HALF + g * 16, 16)]
            i0 = g * 16
            for l in range(16):
                nv = jnp.broadcast_to(nv16[l], (16,))
                for q in range(HID // 16):
                    sl = pl.ds(q * 16, 16)
                    buf[i0 + l, sl] = buf[i0 + l, sl] * nv
            return cc

        lax.fori_loop(0, HALF // 16, grp, 0)
        for j in range(SEGH):
            pltpu.sync_copy(
                buf.at[pl.ds(j * MSEG, MSEG)],
                shared.at[dst_v.at[t * SEGH + j]], add=True)


@functools.partial(
    pl.kernel,
    out_type=jax.ShapeDtypeStruct((NCORE, NNODES, HID), jnp.float32),
    mesh=_mesh,
    compiler_params=_sc_params,
    scratch_types=[
        pltpu.VMEM((EPW // 128, 128), jnp.int32),
        pltpu.VMEM((EPW // 128, 128), jnp.int32),
        pltpu.VMEM((EPW,), jnp.float32),
        pltpu.VMEM((HALF, HID), jnp.float32),
        pltpu.VMEM((HALF, HID), jnp.float32),
        pltpu.VMEM_SHARED((NNODES, HID), jnp.float32),
        pltpu.SemaphoreType.DMA,
        pltpu.SemaphoreType.DMA,
    ],
)
def _sc_messages(ytab_hbm, gidx_hbm, dst_hbm, norm_hbm, zeros_hbm, out_hbm,
                 gidx_v, dst_v, norm_v, rows_a, rows_b, shared, sem_a, sem_b):
    c = lax.axis_index("c")
    w = _wid()
    _striped(lambda off, n: pltpu.sync_copy(
        zeros_hbm.at[pl.ds(off, n)], shared.at[pl.ds(off, n)]))
    base = pl.multiple_of(w * EPW, EPW)
    rbase = pl.multiple_of(base // 128, 8)
    pltpu.sync_copy(gidx_hbm.at[pl.ds(rbase, EPW // 128)], gidx_v)
    pltpu.sync_copy(dst_hbm.at[pl.ds(rbase, EPW // 128)], dst_v)
    pltpu.sync_copy(norm_hbm.at[pl.ds(base, EPW)], norm_v)
    plsc.subcore_barrier()
    _msg_pipeline(ytab_hbm, gidx_v, dst_v, norm_v, rows_a, rows_b,
                  shared, sem_a, sem_b)
    plsc.subcore_barrier()
    _striped(lambda off, n: pltpu.sync_copy(
        shared.at[pl.ds(off, n)], out_hbm.at[c].at[pl.ds(off, n)]))


@functools.partial(
    pl.kernel,
    out_type=[
        jax.ShapeDtypeStruct((NCORE, NNODES, HID), jnp.float32),
        jax.ShapeDtypeStruct((3, 2 * NTGT, HID), jnp.float32),
    ],
    mesh=_mesh,
    compiler_params=_sc_params,
    scratch_types=[
        pltpu.VMEM((EPW // 128, 128), jnp.int32),
        pltpu.VMEM((EPW // 128, 128), jnp.int32),
        pltpu.VMEM((EPW,), jnp.float32),
        pltpu.VMEM((HALF, HID), jnp.float32),
        pltpu.VMEM((HALF, HID), jnp.float32),
        pltpu.VMEM_SHARED((NNODES, HID), jnp.float32),
        pltpu.SemaphoreType.DMA,
        pltpu.SemaphoreType.DMA,
        pltpu.VMEM((128,), jnp.int32),
        pltpu.VMEM((TPW,), jnp.int32),
    ],
)
def _sc_messages_tgt(ytab_hbm, gidx_hbm, dst_hbm, norm_hbm, hr_hbm, tgt_hbm,
                     zeros_hbm, out_hbm, g3_hbm,
                     gidx_v, dst_v, norm_v, rows_a, rows_b, shared,
                     sem_a, sem_b, idxc_v, idxh_v):
    c = lax.axis_index("c")
    s = lax.axis_index("s")
    w = _wid()
    _striped(lambda off, n: pltpu.sync_copy(
        zeros_hbm.at[pl.ds(off, n)], shared.at[pl.ds(off, n)]))
    base = pl.multiple_of(w * EPW, EPW)
    rbase = pl.multiple_of(base // 128, 8)
    pltpu.sync_copy(gidx_hbm.at[pl.ds(rbase, EPW // 128)], gidx_v)
    pltpu.sync_copy(dst_hbm.at[pl.ds(rbase, EPW // 128)], dst_v)
    pltpu.sync_copy(norm_hbm.at[pl.ds(base, EPW)], norm_v)
    plsc.subcore_barrier()
    _msg_pipeline(ytab_hbm, gidx_v, dst_v, norm_v, rows_a, rows_b,
                  shared, sem_a, sem_b)
    plsc.subcore_barrier()
    _striped(lambda off, n: pltpu.sync_copy(
        shared.at[pl.ds(off, n)], out_hbm.at[c].at[pl.ds(off, n)]))
    plsc.subcore_barrier()
    tb = pl.multiple_of(s * 128, 8)
    pltpu.sync_copy(tgt_hbm.at[pl.ds(tb, 128)], idxc_v)
    pltpu.async_copy(
        out_hbm.at[c].at[idxc_v], rows_a.at[pl.ds(0, 128)], sem_a).wait()
    pltpu.sync_copy(rows_a.at[pl.ds(0, 128)], g3_hbm.at[c].at[pl.ds(tb, 128)])
    hb = pl.multiple_of(w * TPW, 8)
    pltpu.sync_copy(tgt_hbm.at[pl.ds(hb, TPW)], idxh_v)
    pltpu.async_copy(hr_hbm.at[idxh_v], rows_b.at[pl.ds(0, TPW)], sem_b).wait()
    pltpu.sync_copy(rows_b.at[pl.ds(0, TPW)], g3_hbm.at[2].at[pl.ds(hb, TPW)])


def _mm_body(a_ref, w_ref, y_ref, r_ref):
    p = jnp.dot(a_ref[...], w_ref[...], preferred_element_type=jnp.float32)
    r_ref[...] = p[:, :HID]
    for r in range(NREL):
        y_ref[r] = p[:, 2 * HID + r * HID:2 * HID + (r + 1) * HID]


def _tc_matmul(h, wcat):
    m, kdim = h.shape
    ncols = wcat.shape[1]
    bm = 400
    return pl.pallas_call(
        _mm_body,
        grid=(m // bm,),
        in_specs=[
            pl.BlockSpec((bm, kdim), lambda i: (i, 0)),
            pl.BlockSpec((kdim, ncols), lambda i: (0, 0)),
        ],
        out_specs=[
            pl.BlockSpec((NREL, bm, HID), lambda i: (0, i, 0)),
            pl.BlockSpec((bm, HID), lambda i: (i, 0)),
        ],
        out_shape=[
            jax.ShapeDtypeStruct((NREL, m, HID), jnp.float32),
            jax.ShapeDtypeStruct((m, HID), jnp.float32),
        ],
    )(h, wcat)


def _mm_fused_body(p0_ref, p1_ref, hr_ref, b_ref, w_ref, y_ref, r_ref):
    a = jnp.maximum(p0_ref[...] + p1_ref[...] + hr_ref[...] + b_ref[...], 0.0)
    p = jnp.dot(a, w_ref[...], preferred_element_type=jnp.float32)
    r_ref[...] = p[:, :HID]
    for r in range(NREL):
        y_ref[r] = p[:, 2 * HID + r * HID:2 * HID + (r + 1) * HID]


def _tc_matmul_fused(p0, p1, hr, bias, wcat):
    m = p0.shape[0]
    ncols = wcat.shape[1]
    bm = 400
    spec = pl.BlockSpec((bm, HID), lambda i: (i, 0))
    return pl.pallas_call(
        _mm_fused_body,
        grid=(m // bm,),
        in_specs=[
            spec, spec, spec,
            pl.BlockSpec((1, HID), lambda i: (0, 0)),
            pl.BlockSpec((HID, ncols), lambda i: (0, 0)),
        ],
        out_specs=[
            pl.BlockSpec((NREL, bm, HID), lambda i: (0, i, 0)),
            pl.BlockSpec((bm, HID), lambda i: (i, 0)),
        ],
        out_shape=[
            jax.ShapeDtypeStruct((NREL, m, HID), jnp.float32),
            jax.ShapeDtypeStruct((m, HID), jnp.float32),
        ],
    )(p0, p1, hr, bias.reshape(1, HID), wcat)


def _dec_body(g_ref, b_ref, w2_ref, s_ref, o_ref):
    h = jnp.maximum(g_ref[0] + g_ref[1] + g_ref[2] + b_ref[...], 0.0)
    x1 = h[:NTGT]
    x2 = h[NTGT:]
    p = jnp.dot(x1, w2_ref[...], preferred_element_type=jnp.float32)
    x2t = jnp.concatenate([x2] * NREL, axis=1)
    rec = jnp.dot(p * x2t, s_ref[...], preferred_element_type=jnp.float32)
    o_ref[...] = jax.nn.sigmoid(rec)


def _tc_decoder(g3, bias, w2cat, sel):
    return pl.pallas_call(
        _dec_body,
        out_shape=jax.ShapeDtypeStruct((NTGT, NREL), jnp.float32),
    )(g3, bias.reshape(1, HID), w2cat, sel)


def kernel(x, edge_index, edge_attr, target_edge_index,
           bases0, comp0, root0, bias0, bases1, comp1, root1, bias1, R, D):
    f32 = jnp.float32
    et = jnp.argmax(edge_attr, axis=1).astype(jnp.int32)
    src = edge_index[0]
    dst = edge_index[1]
    gidx = et * NNODES + src
    pad = EPAD - NEDGE
    gidx_p = jnp.pad(gidx, (0, pad)).reshape(EPAD // 128, 128)
    dst_p = jnp.pad(dst, (0, pad)).reshape(EPAD // 128, 128)
    et_p = jnp.pad(et, (0, pad))
    zeros_n16 = jnp.zeros((NNODES, NREL), f32)
    zeros_nh = jnp.zeros((NNODES, HID), f32)

    c2d = _sc_counts(dst_p, et_p, zeros_n16)
    norm = _sc_norm(c2d, dst_p, et_p)

    def wcat(comp, bases, root):
        indim = root.shape[0]
        w = jnp.einsum('rb,bio->rio', comp, bases)
        w = w.transpose(1, 0, 2).reshape(indim, NREL * HID)
        return jnp.concatenate([root, jnp.zeros((indim, HID), f32), w], axis=1)

    y0, xr0 = _tc_matmul(x, wcat(comp0, bases0, root0))
    ap0 = _sc_messages(y0.reshape(NREL * NNODES, HID), gidx_p, dst_p, norm,
                       zeros_nh)

    y1, hr1 = _tc_matmul_fused(ap0[0], ap0[1], xr0, bias0,
                               wcat(comp1, bases1, root1))
    tgt = target_edge_index.reshape(-1).astype(jnp.int32)
    _, g3 = _sc_messages_tgt(y1.reshape(NREL * NNODES, HID), gidx_p, dst_p,
                             norm, hr1, tgt, zeros_nh)
    w2 = (D[:, :, None] * R[None, :, :] * D[:, None, :])
    w2 = w2.transpose(1, 0, 2).reshape(HID, NREL * HID)
    sel = (lax.broadcasted_iota(jnp.int32, (NREL * HID, NREL), 0) // HID ==
           lax.broadcasted_iota(jnp.int32, (NREL * HID, NREL), 1)).astype(f32)
    return _tc_decoder(g3, bias1, w2, sel)
